# baseline (speedup 1.0000x reference)
import functools

import jax
import jax.numpy as jnp
from jax import lax
from jax.experimental import pallas as pl
from jax.experimental.pallas import tpu as pltpu
from jax.experimental.pallas import tpu_sc as plsc

_NV = 10000
_NRINGS = 4
_NDIRS = 16
_E = _NV * _NRINGS * _NDIRS

_NC = 2
_NS = 16
_NW = _NC * _NS
_CHUNK = _E // _NW
_L = 16


@functools.partial(
    pl.kernel,
    mesh=plsc.VectorSubcoreMesh(core_axis_name="c", subcore_axis_name="s"),
    out_type=jax.ShapeDtypeStruct((2 * _E,), jnp.int32),
    scratch_types=[
        pltpu.VMEM((_CHUNK,), jnp.int32),
        pltpu.VMEM((_CHUNK,), jnp.int32),
        pltpu.VMEM((2 * _CHUNK,), jnp.int32),
    ],
    compiler_params=pltpu.CompilerParams(needs_layout_passes=False),
)
def _interleave_sc(conn_hbm, trans_hbm, out_hbm, conn_v, trans_v, out_v):
    wid = lax.axis_index("s") * _NC + lax.axis_index("c")
    base = wid * _CHUNK
    pltpu.sync_copy(conn_hbm.at[pl.ds(base, _CHUNK)], conn_v)
    pltpu.sync_copy(trans_hbm.at[pl.ds(base, _CHUNK)], trans_v)

    even = 2 * lax.iota(jnp.int32, _L)

    def step(i, _):
        c = conn_v[pl.ds(i * _L, _L)]
        t = trans_v[pl.ds(i * _L, _L)]
        idx = even + (2 * _L) * i
        plsc.store_scatter(out_v, [idx], c)
        plsc.store_scatter(out_v, [idx + 1], t)
        return _

    lax.fori_loop(0, _CHUNK // _L, step, None)
    pltpu.sync_copy(out_v, out_hbm.at[pl.ds(2 * base, 2 * _CHUNK)])


def kernel(inputs, connectivity, transport):
    del inputs
    flat = _interleave_sc(connectivity, transport)
    return flat.reshape(_NV, _NRINGS, _NDIRS, 2)


# speedup vs baseline: 1.0065x; 1.0065x over previous
import functools

import jax
import jax.numpy as jnp
from jax import lax
from jax.experimental import pallas as pl
from jax.experimental.pallas import tpu as pltpu
from jax.experimental.pallas import tpu_sc as plsc

_NV = 10000
_NRINGS = 4
_NDIRS = 16
_E = _NV * _NRINGS * _NDIRS

_NC = 2
_NS = 16
_NW = _NC * _NS
_CHUNK = _E // _NW
_L = 16


@functools.partial(
    pl.kernel,
    mesh=plsc.VectorSubcoreMesh(core_axis_name="c", subcore_axis_name="s"),
    out_type=jax.ShapeDtypeStruct((2 * _E,), jnp.int32),
    scratch_types=[
        pltpu.VMEM((_CHUNK,), jnp.int32),
        pltpu.VMEM((_CHUNK,), jnp.int32),
        pltpu.VMEM((2 * _CHUNK,), jnp.int32),
    ],
    compiler_params=pltpu.CompilerParams(needs_layout_passes=False),
)
def _interleave_sc(conn_hbm, trans_hbm, out_hbm, conn_v, trans_v, out_v):
    wid = lax.axis_index("s") * _NC + lax.axis_index("c")
    base = wid * _CHUNK
    pltpu.sync_copy(conn_hbm.at[pl.ds(base, _CHUNK)], conn_v)
    pltpu.sync_copy(trans_hbm.at[pl.ds(base, _CHUNK)], trans_v)

    even = 2 * lax.iota(jnp.int32, _L)

    @plsc.parallel_loop(0, _CHUNK // _L, unroll=8)
    def _step(i):
        c = conn_v[pl.ds(i * _L, _L)]
        t = trans_v[pl.ds(i * _L, _L)]
        idx = even + (2 * _L) * i
        plsc.store_scatter(out_v, [idx], c)
        plsc.store_scatter(out_v, [idx + 1], t)
    pltpu.sync_copy(out_v, out_hbm.at[pl.ds(2 * base, 2 * _CHUNK)])


def kernel(inputs, connectivity, transport):
    del inputs
    flat = _interleave_sc(connectivity, transport)
    return flat.reshape(_NV, _NRINGS, _NDIRS, 2)


# speedup vs baseline: 1.0165x; 1.0100x over previous
import functools

import jax
import jax.numpy as jnp
from jax import lax
from jax.experimental import pallas as pl
from jax.experimental.pallas import tpu as pltpu
from jax.experimental.pallas import tpu_sc as plsc

_NV = 10000
_NRINGS = 4
_NDIRS = 16
_E = _NV * _NRINGS * _NDIRS

_NC = 2
_NS = 16
_NW = _NC * _NS
_CHUNK = _E // _NW
_L = 16


@functools.partial(
    pl.kernel,
    mesh=plsc.VectorSubcoreMesh(core_axis_name="c", subcore_axis_name="s"),
    out_type=jax.ShapeDtypeStruct((2 * _E,), jnp.int32),
    scratch_types=[
        pltpu.VMEM((_CHUNK,), jnp.int32),
        pltpu.VMEM((_CHUNK,), jnp.int32),
        pltpu.VMEM((2 * _CHUNK,), jnp.int32),
    ],
    compiler_params=pltpu.CompilerParams(needs_layout_passes=False),
)
def _interleave_sc(conn_hbm, trans_hbm, out_hbm, conn_v, trans_v, out_v):
    wid = lax.axis_index("s") * _NC + lax.axis_index("c")
    base = wid * _CHUNK
    pltpu.sync_copy(conn_hbm.at[pl.ds(base, 16)], conn_v.at[pl.ds(0, 16)])
    pltpu.sync_copy(conn_v.at[pl.ds(0, 16)], out_hbm.at[pl.ds(2 * base, 16)])


def kernel(inputs, connectivity, transport):
    del inputs
    flat = _interleave_sc(connectivity, transport)
    return flat.reshape(_NV, _NRINGS, _NDIRS, 2)


# speedup vs baseline: 8.3784x; 8.2424x over previous
import functools

import jax
import jax.numpy as jnp
from jax import lax
from jax.experimental import pallas as pl
from jax.experimental.pallas import tpu as pltpu
from jax.experimental.pallas import tpu_sc as plsc

_NV = 10000
_NR = 4
_ND = 16
_RD = _NR * _ND
_VT = 79
_NVPAD = _VT * 128
_ROWS = _RD * _VT * 2
_L = 16
_NW = 32
_SLAB = 128 * _RD
_TILES_PER_W = 3


@functools.partial(
    pl.kernel,
    mesh=plsc.VectorSubcoreMesh(core_axis_name="c", subcore_axis_name="s"),
    out_type=jax.ShapeDtypeStruct((_ROWS, 128), jnp.int32),
    scratch_types=[
        pltpu.VMEM((_SLAB,), jnp.int32),
        pltpu.VMEM((_SLAB,), jnp.int32),
        pltpu.VMEM((128, 128), jnp.int32),
        pltpu.VMEM((128,), jnp.int32),
        pltpu.SemaphoreType.DMA,
    ],
    compiler_params=pltpu.CompilerParams(needs_layout_passes=False),
)
def _pullback_sc(conn_hbm, trans_hbm, out_hbm, conn_v, trans_v, rows_v,
                 ridx_v, sem):
    wid = lax.axis_index("s") * 2 + lax.axis_index("c")
    lane = lax.iota(jnp.int32, _L)
    lane64 = lane * 64

    for u in range(_TILES_PER_W):
        vt = wid + u * _NW

        @pl.when(vt < _VT)
        def _tile():
            base = vt * _SLAB

            @pl.when(vt < _VT - 1)
            def _stage_full():
                pltpu.sync_copy(conn_hbm.at[pl.ds(base, _SLAB)], conn_v)
                pltpu.sync_copy(trans_hbm.at[pl.ds(base, _SLAB)], trans_v)

            @pl.when(vt == _VT - 1)
            def _stage_tail():
                n_tail = (_NV - (_VT - 1) * 128) * _RD
                pltpu.sync_copy(conn_hbm.at[pl.ds(base, n_tail)],
                                conn_v.at[pl.ds(0, n_tail)])
                pltpu.sync_copy(trans_hbm.at[pl.ds(base, n_tail)],
                                trans_v.at[pl.ds(0, n_tail)])

            for m in range(8):
                k = lane + m * _L
                val = (k >> 1) * (2 * _VT) + (k & 1) + vt * 2
                ridx_v[pl.ds(m * _L, _L)] = val

            for j in range(8):
                idx_j = lane64 + (j * 1024)

                @plsc.parallel_loop(0, _RD, unroll=4)
                def _rd_loop(rd):
                    idx = idx_j + rd
                    c = plsc.load_gather(conn_v, [idx])
                    t = plsc.load_gather(trans_v, [idx])
                    rows_v[2 * rd, pl.ds(j * _L, _L)] = c
                    rows_v[2 * rd + 1, pl.ds(j * _L, _L)] = t

            pltpu.async_copy(rows_v, out_hbm.at[ridx_v], sem).wait()


def kernel(inputs, connectivity, transport):
    del inputs
    z2 = _pullback_sc(connectivity, transport)
    a = z2.reshape(_NR, _ND, _VT, 2, 128)
    y = a.transpose(2, 4, 0, 1, 3).reshape(_NVPAD, _NR, _ND, 2)
    return y[:_NV]


# speedup vs baseline: 11.1380x; 1.3294x over previous
import functools

import jax
import jax.numpy as jnp
from jax import lax
from jax.experimental import pallas as pl
from jax.experimental.pallas import tpu as pltpu
from jax.experimental.pallas import tpu_sc as plsc

_NV = 10000
_NR = 4
_ND = 16
_RD = _NR * _ND
_VT = 79
_NVPAD = _VT * 128
_ROWS = _RD * _VT * 2
_L = 16
_NW = 32
_SLAB = 128 * _RD
_TILES_PER_W = 3


@functools.partial(
    pl.kernel,
    mesh=plsc.VectorSubcoreMesh(core_axis_name="c", subcore_axis_name="s"),
    out_type=jax.ShapeDtypeStruct((_ROWS, 128), jnp.int32),
    scratch_types=[
        pltpu.VMEM((_SLAB,), jnp.int32),
        pltpu.VMEM((_SLAB,), jnp.int32),
        pltpu.VMEM((_SLAB,), jnp.int32),
        pltpu.VMEM((_SLAB,), jnp.int32),
        pltpu.VMEM((128, 128), jnp.int32),
        pltpu.VMEM((128,), jnp.int32),
        pltpu.SemaphoreType.DMA,
    ],
    compiler_params=pltpu.CompilerParams(needs_layout_passes=False),
)
def _pullback_sc(conn_hbm, trans_hbm, out_hbm, conn_v, trans_v, rotc_v,
                 rott_v, rows_v, ridx_v, sem):
    wid = lax.axis_index("s") * 2 + lax.axis_index("c")
    lane = lax.iota(jnp.int32, _L)
    lane64 = lane * 64

    for u in range(_TILES_PER_W):
        vt = wid + u * _NW

        @pl.when(vt < _VT)
        def _tile():
            base = vt * _SLAB

            @pl.when(vt < _VT - 1)
            def _stage_full():
                pltpu.sync_copy(conn_hbm.at[pl.ds(base, _SLAB)], conn_v)
                pltpu.sync_copy(trans_hbm.at[pl.ds(base, _SLAB)], trans_v)

            @pl.when(vt == _VT - 1)
            def _stage_tail():
                n_tail = (_NV - (_VT - 1) * 128) * _RD
                pltpu.sync_copy(conn_hbm.at[pl.ds(base, n_tail)],
                                conn_v.at[pl.ds(0, n_tail)])
                pltpu.sync_copy(trans_hbm.at[pl.ds(base, n_tail)],
                                trans_v.at[pl.ds(0, n_tail)])

            for m in range(8):
                k = lane + m * _L
                val = (k >> 1) * (2 * _VT) + (k & 1) + vt * 2
                ridx_v[pl.ds(m * _L, _L)] = val

            for rd0 in range(0, _RD, _L):

                @plsc.parallel_loop(0, 128, unroll=4)
                def _rot_loop(v):
                    c = conn_v[pl.ds(v * _RD + rd0, _L)]
                    t = trans_v[pl.ds(v * _RD + rd0, _L)]
                    idx = ((lane + (rd0 + v)) & 63) + v * _RD
                    plsc.store_scatter(rotc_v, [idx], c)
                    plsc.store_scatter(rott_v, [idx], t)

            for j in range(8):
                lane64j = lane64 + j * (_L * _RD)

                @plsc.parallel_loop(0, _RD, unroll=4)
                def _rd_loop(rd):
                    rot = (lane + (rd + j * _L)) & 63
                    idx = lane64j + rot
                    c = plsc.load_gather(rotc_v, [idx])
                    t = plsc.load_gather(rott_v, [idx])
                    rows_v[2 * rd, pl.ds(j * _L, _L)] = c
                    rows_v[2 * rd + 1, pl.ds(j * _L, _L)] = t

            pltpu.async_copy(rows_v, out_hbm.at[ridx_v], sem).wait()


def kernel(inputs, connectivity, transport):
    del inputs
    z2 = _pullback_sc(connectivity, transport)
    a = z2.reshape(_NR, _ND, _VT, 2, 128)
    y = a.transpose(2, 4, 0, 1, 3).reshape(_NVPAD, _NR, _ND, 2)
    return y[:_NV]
